# baseline (speedup 1.0000x reference)
import functools

import jax
import jax.numpy as jnp
from jax import lax
from jax.experimental import pallas as pl
from jax.experimental.pallas import tpu as pltpu
from jax.experimental.pallas import tpu_sc as plsc

B, L = 4096, 200
V, D = 1000, 64
VP = 1024
KS = VP // 128
T = 100

NC, NS = 2, 16
NW = NC * NS
S = 2
RS = B // S
ROWS_PER_W = RS // NW
CH = 32
NCH = ROWS_PER_W // CH
NCHUNKS = RS // CH

UNROLL = 8


def _hist_body(sent_hbm, counts_hbm, sent_a, sent_b, cnt_a, cnt_b,
               sem_a, sem_b):
    wid = lax.axis_index("s") * NC + lax.axis_index("c")
    lanes = lax.iota(jnp.int32, 16)
    zeros16 = jnp.zeros((16,), jnp.float32)
    zeros_i = jnp.zeros((16,), jnp.int32)
    ones16 = jnp.ones((16,), jnp.float32)

    def zero_buf(cnt):
        @plsc.parallel_loop(0, KS * CH, unroll=4)
        def _zbody(r):
            for j in range(8):
                cnt[r, pl.ds(j * 16, 16)] = zeros16

    zero_buf(cnt_a)
    zero_buf(cnt_b)

    def sweep(sent, cnt, op):
        for g in range(CH // 16):
            row = g * 16 + lanes
            rowoff_s = row * L

            @plsc.parallel_loop(0, L, unroll=UNROLL)
            def _lbody(l):
                col = plsc.load_gather(sent, [zeros_i, rowoff_s + l])
                ridx = ((col >> 7) << 5) + row
                cidx = col & 127
                if op == "add":
                    plsc.addupdate_scatter(cnt, [ridx, cidx], ones16)
                else:
                    plsc.store_scatter(cnt, [ridx, cidx], zeros16)

    bufs = [(sent_a, cnt_a, sem_a), (sent_b, cnt_b, sem_b)]
    for c in range(NCH):
        sent, cnt, sem = bufs[c % 2]
        chunk = wid * NCH + c
        if c >= 2:
            pltpu.make_async_copy(cnt.reshape(KS, CH, 128),
                                  counts_hbm.at[chunk - 2], sem).wait()
            sweep(sent, cnt, "zero")
        pltpu.sync_copy(sent_hbm.at[chunk], sent.at[0])
        sweep(sent, cnt, "add")
        pltpu.async_copy(cnt.reshape(KS, CH, 128), counts_hbm.at[chunk], sem)
    for c in range(max(0, NCH - 2), NCH):
        sent, cnt, sem = bufs[c % 2]
        chunk = wid * NCH + c
        pltpu.make_async_copy(cnt.reshape(KS, CH, 128),
                              counts_hbm.at[chunk], sem).wait()


@functools.cache
def _make_hist():
    mesh = plsc.VectorSubcoreMesh(core_axis_name="c", subcore_axis_name="s")
    return functools.partial(
        pl.kernel,
        mesh=mesh,
        out_type=jax.ShapeDtypeStruct((NCHUNKS, KS, CH, 128), jnp.float32),
        scratch_types=[
            pltpu.VMEM((1, CH * L), jnp.int32),
            pltpu.VMEM((1, CH * L), jnp.int32),
            pltpu.VMEM((KS * CH, 128), jnp.float32),
            pltpu.VMEM((KS * CH, 128), jnp.float32),
            pltpu.SemaphoreType.DMA,
            pltpu.SemaphoreType.DMA,
        ],
        compiler_params=pltpu.CompilerParams(needs_layout_passes=False),
    )(_hist_body)


BB = 512
CB = BB // CH


def _tc_body(counts_ref, table_ref, w_ref, b_ref, out_ref):
    counts = counts_ref[...]
    bow = None
    for k in range(KS):
        lhs = counts[:, k].reshape(BB, 128).astype(jnp.bfloat16)
        part = jnp.dot(lhs, table_ref[k].astype(jnp.bfloat16),
                       preferred_element_type=jnp.float32)
        bow = part if bow is None else bow + part
    sig = 1.0 / (1.0 + jnp.exp(-bow))
    tag = lax.dot_general(sig, w_ref[...], (((1,), (1,)), ((), ())),
                          preferred_element_type=jnp.float32)
    out_ref[...] = tag + b_ref[...]


def _tc_call(counts, table2, w, b2d):
    return pl.pallas_call(
        _tc_body,
        grid=(RS // BB,),
        in_specs=[
            pl.BlockSpec((CB, KS, CH, 128), lambda i: (i, 0, 0, 0)),
            pl.BlockSpec((KS, 128, D), lambda i: (0, 0, 0)),
            pl.BlockSpec((T, D), lambda i: (0, 0)),
            pl.BlockSpec((1, T), lambda i: (0, 0)),
        ],
        out_specs=pl.BlockSpec((BB, T), lambda i: (i, 0)),
        out_shape=jax.ShapeDtypeStruct((RS, T), jnp.float32),
    )(counts, table2, w, b2d)


def kernel(sentence, emb_table, W, b):
    sent_rows = sentence.astype(jnp.int32).reshape(S, NCHUNKS, CH * L)
    table2 = jnp.pad(emb_table, ((0, VP - V), (0, 0))).reshape(KS, 128, D)
    b2d = b.reshape(1, T)
    hist = _make_hist()
    counts = [hist(sent_rows[s]) for s in range(S)]
    outs = [_tc_call(counts[s], table2, W, b2d) for s in range(S)]
    return jnp.concatenate(outs, axis=0)


# speedup vs baseline: 1.1576x; 1.1576x over previous
import functools

import jax
import jax.numpy as jnp
from jax import lax
from jax.experimental import pallas as pl
from jax.experimental.pallas import tpu as pltpu
from jax.experimental.pallas import tpu_sc as plsc

B, L = 4096, 200
V, D = 1000, 64
VP = 1024
KS = VP // 256
T = 100

NC, NS = 2, 16
NW = NC * NS
ROWS_PER_W = B // NW
CH = 32
NCH = ROWS_PER_W // CH
NCHUNKS = B // CH

UNROLL = 8


def _hist_body(sent_hbm, counts_hbm, sent_a, sent_b, cnt_a, cnt_b,
               sem_a, sem_b):
    wid = lax.axis_index("s") * NC + lax.axis_index("c")
    lanes = lax.iota(jnp.int32, 16)
    zeros16 = jnp.zeros((16,), jnp.int32)
    zeros_i = jnp.zeros((16,), jnp.int32)
    ones16 = jnp.ones((16,), jnp.int32)

    def zero_buf(cnt):
        @plsc.parallel_loop(0, KS * CH, unroll=4)
        def _zbody(r):
            for j in range(8):
                cnt[r, pl.ds(j * 16, 16)] = zeros16

    zero_buf(cnt_a)
    zero_buf(cnt_b)

    def sweep(sent, cnt, op):
        for g in range(CH // 16):
            row = g * 16 + lanes
            rowoff_s = row * L

            @plsc.parallel_loop(0, L, unroll=UNROLL)
            def _lbody(l):
                col = plsc.load_gather(sent, [zeros_i, rowoff_s + l])
                ridx = ((col >> 8) << 5) + row
                cidx = (col >> 1) & 127
                if op == "add":
                    val = ones16 << ((col & 1) << 4)
                    plsc.addupdate_scatter(cnt, [ridx, cidx], val)
                else:
                    plsc.store_scatter(cnt, [ridx, cidx], zeros16)

    bufs = [(sent_a, cnt_a, sem_a), (sent_b, cnt_b, sem_b)]
    for c in range(NCH):
        sent, cnt, sem = bufs[c % 2]
        chunk = wid * NCH + c
        if c >= 2:
            pltpu.make_async_copy(cnt.reshape(KS, CH, 128),
                                  counts_hbm.at[chunk - 2], sem).wait()
            sweep(sent, cnt, "zero")
        pltpu.sync_copy(sent_hbm.at[chunk], sent.at[0])
        sweep(sent, cnt, "add")
        pltpu.async_copy(cnt.reshape(KS, CH, 128), counts_hbm.at[chunk], sem)
    for c in (NCH - 2, NCH - 1):
        sent, cnt, sem = bufs[c % 2]
        chunk = wid * NCH + c
        pltpu.make_async_copy(cnt.reshape(KS, CH, 128),
                              counts_hbm.at[chunk], sem).wait()


@functools.cache
def _make_hist():
    mesh = plsc.VectorSubcoreMesh(core_axis_name="c", subcore_axis_name="s")
    return functools.partial(
        pl.kernel,
        mesh=mesh,
        out_type=jax.ShapeDtypeStruct((NCHUNKS, KS, CH, 128), jnp.int32),
        scratch_types=[
            pltpu.VMEM((1, CH * L), jnp.int32),
            pltpu.VMEM((1, CH * L), jnp.int32),
            pltpu.VMEM((KS * CH, 128), jnp.int32),
            pltpu.VMEM((KS * CH, 128), jnp.int32),
            pltpu.SemaphoreType.DMA,
            pltpu.SemaphoreType.DMA,
        ],
        compiler_params=pltpu.CompilerParams(needs_layout_passes=False),
    )(_hist_body)


BB = 512
CB = BB // CH


def _tc_body(counts_ref, table_ref, w_ref, b_ref, out_ref):
    counts = counts_ref[...]
    bow = None
    for k in range(KS):
        w = counts[:, k].reshape(BB, 128)
        lo = (w & 0xFFFF).astype(jnp.bfloat16)
        hi = lax.shift_right_logical(w, 16).astype(jnp.bfloat16)
        tbl = table_ref[k]
        part = jnp.dot(lo, tbl[:, 0].astype(jnp.bfloat16),
                       preferred_element_type=jnp.float32)
        part += jnp.dot(hi, tbl[:, 1].astype(jnp.bfloat16),
                        preferred_element_type=jnp.float32)
        bow = part if bow is None else bow + part
    sig = 1.0 / (1.0 + jnp.exp(-bow))
    tag = lax.dot_general(sig, w_ref[...], (((1,), (1,)), ((), ())),
                          preferred_element_type=jnp.float32)
    out_ref[...] = tag + b_ref[...]


def _tc_call(counts, table2, w, b2d):
    return pl.pallas_call(
        _tc_body,
        grid=(B // BB,),
        in_specs=[
            pl.BlockSpec((CB, KS, CH, 128), lambda i: (i, 0, 0, 0)),
            pl.BlockSpec((KS, 128, 2, D), lambda i: (0, 0, 0, 0)),
            pl.BlockSpec((T, D), lambda i: (0, 0)),
            pl.BlockSpec((1, T), lambda i: (0, 0)),
        ],
        out_specs=pl.BlockSpec((BB, T), lambda i: (i, 0)),
        out_shape=jax.ShapeDtypeStruct((B, T), jnp.float32),
    )(counts, table2, w, b2d)


def kernel(sentence, emb_table, W, b):
    sent_rows = sentence.astype(jnp.int32).reshape(NCHUNKS, CH * L)
    counts = _make_hist()(sent_rows)
    table2 = jnp.pad(emb_table, ((0, VP - V), (0, 0))).reshape(KS, 128, 2, D)
    return _tc_call(counts, table2, W, b.reshape(1, T))
